# baseline (speedup 1.0000x reference)
import functools

import jax
import jax.numpy as jnp
from jax import lax
from jax.experimental import pallas as pl
from jax.experimental.pallas import tpu as pltpu
from jax.experimental.pallas import tpu_sc as plsc

N = 10000
E = 320000
DIN = 128
HID = 128
DOUT = 64

NC = 2
NS = 16
NW = NC * NS
FD = HID // NC
K = 80
EWD = E // NW
CHD = EWD // K
EWP = E // NS
CHP = EWP // K
RZC = 80
NRC = N // RZC
NZJ = -(-NRC // NS)

BT = 1000
GT = N // BT


def _mesh():
    return plsc.VectorSubcoreMesh(
        core_axis_name="c", subcore_axis_name="s", num_cores=NC, num_subcores=NS
    )


def _sc_degree(dst3):
    @functools.partial(
        pl.kernel,
        out_type=jax.ShapeDtypeStruct((NC, N, 16), jnp.float32),
        mesh=_mesh(),
        scratch_types=[
            pltpu.VMEM((CHD, K), jnp.int32),
            pltpu.VMEM((K, 16), jnp.float32),
            pltpu.VMEM((RZC, 16), jnp.float32),
            pltpu.VMEM_SHARED((N, 16), jnp.float32),
            pltpu.SemaphoreType.DMA,
        ],
        compiler_params=pltpu.CompilerParams(use_tc_tiling_on_sc=False),
    )
    def deg_kernel(dst_hbm, out_hbm, didx_v, ones_v, zbuf_v, acc_sh, isem):
        cid = lax.axis_index("c")
        sid = lax.axis_index("s")
        wid = cid * NS + sid
        di = pltpu.async_copy(dst_hbm.at[wid], didx_v, isem)

        def fill(i, carry):
            ones_v[i, :] = jnp.full((16,), 1.0, jnp.float32)
            return carry

        lax.fori_loop(0, K, fill, 0)

        def zrow(i, carry):
            zbuf_v[i, :] = jnp.zeros((16,), jnp.float32)
            return carry

        lax.fori_loop(0, RZC, zrow, 0)

        def zchunk(j, carry):
            c = sid + j * NS

            @pl.when(c < NRC)
            def _():
                pltpu.sync_copy(zbuf_v, acc_sh.at[pl.ds(c * RZC, RZC)])

            return carry

        lax.fori_loop(0, NZJ, zchunk, 0)
        di.wait()
        plsc.subcore_barrier()

        def chunk(c, carry):
            pltpu.sync_copy(ones_v, acc_sh.at[didx_v.at[c]], add=True)
            return carry

        lax.fori_loop(0, CHD, chunk, 0)
        plsc.subcore_barrier()

        def ochunk(j, carry):
            c = sid + j * NS

            @pl.when(c < NRC)
            def _():
                r0 = c * RZC
                pltpu.sync_copy(acc_sh.at[pl.ds(r0, RZC)],
                                out_hbm.at[cid, pl.ds(r0, RZC)])

            return carry

        lax.fori_loop(0, NZJ, ochunk, 0)

    return deg_kernel(dst3)


def _sc_propagate(hsplit, src3, dst3):
    @functools.partial(
        pl.kernel,
        out_type=jax.ShapeDtypeStruct((NC, N, FD), jnp.float32),
        mesh=_mesh(),
        scratch_types=[
            pltpu.VMEM((CHP, K), jnp.int32),
            pltpu.VMEM((CHP, K), jnp.int32),
            [pltpu.VMEM((K, FD), jnp.float32)] * 9,
            pltpu.VMEM_SHARED((N, FD), jnp.float32),
            [pltpu.SemaphoreType.DMA] * 9,
            [pltpu.SemaphoreType.DMA] * 9,
        ],
        compiler_params=pltpu.CompilerParams(use_tc_tiling_on_sc=False),
    )
    def prop_kernel(hs_hbm, src_hbm, dst_hbm, out_hbm,
                    sidx_v, didx_v, rows_bufs, acc_sh, gsems, ssems):
        zbuf_v = rows_bufs[0]
        cid = lax.axis_index("c")
        sid = lax.axis_index("s")
        dsi = pltpu.async_copy(src_hbm.at[sid], sidx_v, gsems[0])
        ddi = pltpu.async_copy(dst_hbm.at[sid], didx_v, gsems[1])
        table = hs_hbm.at[cid]

        def zrow(i, carry):
            for j in range(FD // 16):
                zbuf_v[i, pl.ds(j * 16, 16)] = jnp.zeros((16,), jnp.float32)
            return carry

        lax.fori_loop(0, RZC, zrow, 0)

        def zchunk(j, carry):
            c = sid + j * NS

            @pl.when(c < NRC)
            def _():
                pltpu.sync_copy(zbuf_v, acc_sh.at[pl.ds(c * RZC, RZC)])

            return carry

        lax.fori_loop(0, NZJ, zchunk, 0)
        dsi.wait()
        ddi.wait()
        plsc.subcore_barrier()

        NB = len(rows_bufs)
        NG = CHP // NB
        TAIL = CHP - NG * NB

        for j in range(NB):
            pltpu.async_copy(table.at[sidx_v.at[j]], rows_bufs[j], gsems[j])

        def chunk_group(g, carry):
            c = g * NB
            sdescs = []
            for j in range(NB):
                pltpu.make_async_copy(table.at[sidx_v.at[c + j]],
                                      rows_bufs[j], gsems[j]).wait()
                sdescs.append(
                    pltpu.async_copy(rows_bufs[j],
                                     acc_sh.at[didx_v.at[c + j]],
                                     ssems[j], add=True))
            for j in range(NB):
                sdescs[j].wait()

                @pl.when(g < NG - 1)
                def _():
                    pltpu.async_copy(table.at[sidx_v.at[c + NB + j]],
                                     rows_bufs[j], gsems[j])

            return carry

        lax.fori_loop(0, NG, chunk_group, 0)
        for t in range(TAIL):
            c = NG * NB + t
            pltpu.async_copy(table.at[sidx_v.at[c]], rows_bufs[t],
                             gsems[t]).wait()
            pltpu.sync_copy(rows_bufs[t], acc_sh.at[didx_v.at[c]], add=True)
        plsc.subcore_barrier()

        def ochunk(j, carry):
            c = sid + j * NS

            @pl.when(c < NRC)
            def _():
                r0 = c * RZC
                pltpu.sync_copy(acc_sh.at[pl.ds(r0, RZC)],
                                out_hbm.at[cid, pl.ds(r0, RZC)])

            return carry

        lax.fori_loop(0, NZJ, ochunk, 0)

    return prop_kernel(hsplit, src3, dst3)


def _tc_pre(degp, x, W1):
    def body(degp_ref, x_ref, w_ref, dinv_ref, hs_ref):
        deg = degp_ref[0, :, 0:1] + degp_ref[1, :, 0:1] + 1.0
        dinv = lax.rsqrt(jnp.maximum(deg, 1.0))
        dinvb = jnp.broadcast_to(dinv, (BT, HID))
        dinv_ref[...] = dinvb
        t = (
            jnp.dot(x_ref[...], w_ref[...], preferred_element_type=jnp.float32)
            * dinvb
        )
        hs_ref[0] = t[:, :FD]
        hs_ref[1] = t[:, FD:]

    return pl.pallas_call(
        body,
        grid=(GT,),
        in_specs=[
            pl.BlockSpec((NC, BT, 16), lambda i: (0, i, 0)),
            pl.BlockSpec((BT, DIN), lambda i: (i, 0)),
            pl.BlockSpec((DIN, HID), lambda i: (0, 0)),
        ],
        out_specs=[
            pl.BlockSpec((BT, HID), lambda i: (i, 0)),
            pl.BlockSpec((NC, BT, FD), lambda i: (0, i, 0)),
        ],
        out_shape=[
            jax.ShapeDtypeStruct((N, HID), jnp.float32),
            jax.ShapeDtypeStruct((NC, N, FD), jnp.float32),
        ],
    )(degp, x, W1)


def _tc_mid(accf, hsprev, dinv, b2d, Wn):
    def body(accf_ref, hs_ref, dinv_ref, b_ref, w_ref, h_ref, hn_ref):
        cat = jnp.concatenate(
            [accf_ref[0] + hs_ref[0], accf_ref[1] + hs_ref[1]], axis=1
        )
        t = cat * dinv_ref[...] + b_ref[...]
        h = jnp.maximum(t, 0.0)
        h_ref[...] = h
        t2 = (
            jnp.dot(h, w_ref[...], preferred_element_type=jnp.float32)
            * dinv_ref[...]
        )
        hn_ref[0] = t2[:, :FD]
        hn_ref[1] = t2[:, FD:]

    return pl.pallas_call(
        body,
        grid=(GT,),
        in_specs=[
            pl.BlockSpec((NC, BT, FD), lambda i: (0, i, 0)),
            pl.BlockSpec((NC, BT, FD), lambda i: (0, i, 0)),
            pl.BlockSpec((BT, HID), lambda i: (i, 0)),
            pl.BlockSpec((1, HID), lambda i: (0, 0)),
            pl.BlockSpec((HID, HID), lambda i: (0, 0)),
        ],
        out_specs=[
            pl.BlockSpec((BT, HID), lambda i: (i, 0)),
            pl.BlockSpec((NC, BT, FD), lambda i: (0, i, 0)),
        ],
        out_shape=[
            jax.ShapeDtypeStruct((N, HID), jnp.float32),
            jax.ShapeDtypeStruct((NC, N, FD), jnp.float32),
        ],
    )(accf, hsprev, dinv, b2d, Wn)


def _tc_jkpart(h1, h2, Wl, bl2d):
    def body(h1_ref, h2_ref, wl_ref, bl_ref, out_ref):
        acc = jnp.dot(h1_ref[...], wl_ref[0:HID, :],
                      preferred_element_type=jnp.float32)
        acc += jnp.dot(h2_ref[...], wl_ref[HID:2 * HID, :],
                       preferred_element_type=jnp.float32)
        out_ref[...] = acc + bl_ref[...]

    return pl.pallas_call(
        body,
        grid=(GT,),
        in_specs=[
            pl.BlockSpec((BT, HID), lambda i: (i, 0)),
            pl.BlockSpec((BT, HID), lambda i: (i, 0)),
            pl.BlockSpec((3 * HID, DOUT), lambda i: (0, 0)),
            pl.BlockSpec((1, DOUT), lambda i: (0, 0)),
        ],
        out_specs=pl.BlockSpec((BT, DOUT), lambda i: (i, 0)),
        out_shape=jax.ShapeDtypeStruct((N, DOUT), jnp.float32),
    )(h1, h2, Wl, bl2d)


def _tc_fin(accf, hs3, dinv, b2d, part, Wl):
    def body(accf_ref, hs_ref, dinv_ref, b_ref, part_ref, wl_ref, out_ref):
        cat = jnp.concatenate(
            [accf_ref[0] + hs_ref[0], accf_ref[1] + hs_ref[1]], axis=1
        )
        t = cat * dinv_ref[...] + b_ref[...]
        h3 = jnp.maximum(t, 0.0)
        out_ref[...] = part_ref[...] + jnp.dot(
            h3, wl_ref[2 * HID:3 * HID, :],
            preferred_element_type=jnp.float32)

    return pl.pallas_call(
        body,
        grid=(GT,),
        in_specs=[
            pl.BlockSpec((NC, BT, FD), lambda i: (0, i, 0)),
            pl.BlockSpec((NC, BT, FD), lambda i: (0, i, 0)),
            pl.BlockSpec((BT, HID), lambda i: (i, 0)),
            pl.BlockSpec((1, HID), lambda i: (0, 0)),
            pl.BlockSpec((BT, DOUT), lambda i: (i, 0)),
            pl.BlockSpec((3 * HID, DOUT), lambda i: (0, 0)),
        ],
        out_specs=pl.BlockSpec((BT, DOUT), lambda i: (i, 0)),
        out_shape=jax.ShapeDtypeStruct((N, DOUT), jnp.float32),
    )(accf, hs3, dinv, b2d, part, Wl)


def kernel(x, edge_index, W1, b1, W2, b2, W3, b3, Wl, bl):
    srcd = edge_index[0].reshape(NW, CHD, K)
    dstd = edge_index[1].reshape(NW, CHD, K)
    srcp = edge_index[0].reshape(NS, CHP, K)
    dstp = edge_index[1].reshape(NS, CHP, K)

    degp = _sc_degree(dstd)
    dinv, hs1 = _tc_pre(degp, x, W1)

    acc1 = _sc_propagate(hs1, srcp, dstp)
    h1, hs2 = _tc_mid(acc1, hs1, dinv, b1.reshape(1, HID), W2)

    acc2 = _sc_propagate(hs2, srcp, dstp)
    h2, hs3 = _tc_mid(acc2, hs2, dinv, b2.reshape(1, HID), W3)

    acc3 = _sc_propagate(hs3, srcp, dstp)
    part = _tc_jkpart(h1, h2, Wl, bl.reshape(1, DOUT))
    out = _tc_fin(acc3, hs3, dinv, b3.reshape(1, HID), part, Wl)
    return out


# speedup vs baseline: 1.0180x; 1.0180x over previous
import functools

import jax
import jax.numpy as jnp
from jax import lax
from jax.experimental import pallas as pl
from jax.experimental.pallas import tpu as pltpu
from jax.experimental.pallas import tpu_sc as plsc

N = 10000
E = 320000
DIN = 128
HID = 128
DOUT = 64

NC = 2
NS = 16
NW = NC * NS
K = 80
EW = E // NW
CH = EW // K
CI = 63
NB = 3
RZC = 80
NRC = N // RZC
NZJ = -(-NRC // NS)

BT = 1000
GT = N // BT


def _mesh():
    return plsc.VectorSubcoreMesh(
        core_axis_name="c", subcore_axis_name="s", num_cores=NC, num_subcores=NS
    )


def _sc_degree(dst3):
    @functools.partial(
        pl.kernel,
        out_type=jax.ShapeDtypeStruct((NC, N, 16), jnp.float32),
        mesh=_mesh(),
        scratch_types=[
            pltpu.VMEM((CH, K), jnp.int32),
            pltpu.VMEM((K, 16), jnp.float32),
            pltpu.VMEM((RZC, 16), jnp.float32),
            pltpu.VMEM_SHARED((N, 16), jnp.float32),
            pltpu.SemaphoreType.DMA,
        ],
        compiler_params=pltpu.CompilerParams(use_tc_tiling_on_sc=False),
    )
    def deg_kernel(dst_hbm, out_hbm, didx_v, ones_v, zbuf_v, acc_sh, isem):
        cid = lax.axis_index("c")
        sid = lax.axis_index("s")
        wid = cid * NS + sid
        di = pltpu.async_copy(dst_hbm.at[wid], didx_v, isem)

        def fill(i, carry):
            ones_v[i, :] = jnp.full((16,), 1.0, jnp.float32)
            return carry

        lax.fori_loop(0, K, fill, 0)

        def zrow(i, carry):
            zbuf_v[i, :] = jnp.zeros((16,), jnp.float32)
            return carry

        lax.fori_loop(0, RZC, zrow, 0)

        def zchunk(j, carry):
            c = sid + j * NS

            @pl.when(c < NRC)
            def _():
                pltpu.sync_copy(zbuf_v, acc_sh.at[pl.ds(c * RZC, RZC)])

            return carry

        lax.fori_loop(0, NZJ, zchunk, 0)
        di.wait()
        plsc.subcore_barrier()

        def chunk(c, carry):
            pltpu.sync_copy(ones_v, acc_sh.at[didx_v.at[c]], add=True)
            return carry

        lax.fori_loop(0, CH, chunk, 0)
        plsc.subcore_barrier()

        def ochunk(j, carry):
            c = sid + j * NS

            @pl.when(c < NRC)
            def _():
                r0 = c * RZC
                pltpu.sync_copy(acc_sh.at[pl.ds(r0, RZC)],
                                out_hbm.at[cid, pl.ds(r0, RZC)])

            return carry

        lax.fori_loop(0, NZJ, ochunk, 0)

    return deg_kernel(dst3)


def _sc_propagate(hhat, src3, dst3):
    @functools.partial(
        pl.kernel,
        out_type=jax.ShapeDtypeStruct((NC, N, HID), jnp.float32),
        mesh=_mesh(),
        scratch_types=[
            pltpu.VMEM((CI, K), jnp.int32),
            pltpu.VMEM((CI, K), jnp.int32),
            [pltpu.VMEM((K, HID), jnp.float32)] * NB,
            pltpu.VMEM_SHARED((N, HID), jnp.float32),
            [pltpu.SemaphoreType.DMA] * NB,
            [pltpu.SemaphoreType.DMA] * NB,
        ],
        compiler_params=pltpu.CompilerParams(use_tc_tiling_on_sc=False),
    )
    def prop_kernel(hhat_hbm, src_hbm, dst_hbm, out_hbm,
                    sidx_v, didx_v, rows_bufs, acc_sh, gsems, ssems):
        zbuf_v = rows_bufs[0]
        cid = lax.axis_index("c")
        sid = lax.axis_index("s")
        wid = cid * NS + sid
        dsi = pltpu.async_copy(src_hbm.at[wid, pl.ds(0, CI)], sidx_v, gsems[0])
        ddi = pltpu.async_copy(dst_hbm.at[wid, pl.ds(0, CI)], didx_v, gsems[1])

        def zrow(i, carry):
            for j in range(HID // 16):
                zbuf_v[i, pl.ds(j * 16, 16)] = jnp.zeros((16,), jnp.float32)
            return carry

        lax.fori_loop(0, RZC, zrow, 0)

        def zchunk(j, carry):
            c = sid + j * NS

            @pl.when(c < NRC)
            def _():
                pltpu.sync_copy(zbuf_v, acc_sh.at[pl.ds(c * RZC, RZC)])

            return carry

        lax.fori_loop(0, NZJ, zchunk, 0)
        dsi.wait()
        ddi.wait()
        plsc.subcore_barrier()

        def run_phase(count):
            ng = count // NB
            tail = count - ng * NB

            for j in range(NB):
                pltpu.async_copy(hhat_hbm.at[sidx_v.at[j]], rows_bufs[j],
                                 gsems[j])

            def chunk_group(g, carry):
                c = g * NB
                sdescs = []
                for j in range(NB):
                    pltpu.make_async_copy(hhat_hbm.at[sidx_v.at[c + j]],
                                          rows_bufs[j], gsems[j]).wait()
                    sdescs.append(
                        pltpu.async_copy(rows_bufs[j],
                                         acc_sh.at[didx_v.at[c + j]],
                                         ssems[j], add=True))
                for j in range(NB):
                    sdescs[j].wait()

                    @pl.when(g < ng - 1)
                    def _():
                        pltpu.async_copy(hhat_hbm.at[sidx_v.at[c + NB + j]],
                                         rows_bufs[j], gsems[j])

                return carry

            lax.fori_loop(0, ng, chunk_group, 0)
            for t in range(tail):
                c = ng * NB + t
                pltpu.async_copy(hhat_hbm.at[sidx_v.at[c]], rows_bufs[t],
                                 gsems[t]).wait()
                pltpu.sync_copy(rows_bufs[t], acc_sh.at[didx_v.at[c]],
                                add=True)

        run_phase(CI)
        pltpu.sync_copy(src_hbm.at[wid, pl.ds(CI, CH - CI)],
                        sidx_v.at[pl.ds(0, CH - CI)])
        pltpu.sync_copy(dst_hbm.at[wid, pl.ds(CI, CH - CI)],
                        didx_v.at[pl.ds(0, CH - CI)])
        run_phase(CH - CI)
        plsc.subcore_barrier()

        def ochunk(j, carry):
            c = sid + j * NS

            @pl.when(c < NRC)
            def _():
                r0 = c * RZC
                pltpu.sync_copy(acc_sh.at[pl.ds(r0, RZC)],
                                out_hbm.at[cid, pl.ds(r0, RZC)])

            return carry

        lax.fori_loop(0, NZJ, ochunk, 0)

    return prop_kernel(hhat, src3, dst3)


def _tc_pre(degp, x, W1):
    def body(degp_ref, x_ref, w_ref, dinv_ref, hhat_ref):
        deg = degp_ref[0, :, 0:1] + degp_ref[1, :, 0:1] + 1.0
        dinv = lax.rsqrt(jnp.maximum(deg, 1.0))
        dinvb = jnp.broadcast_to(dinv, (BT, HID))
        dinv_ref[...] = dinvb
        hhat_ref[...] = (
            jnp.dot(x_ref[...], w_ref[...], preferred_element_type=jnp.float32)
            * dinvb
        )

    return pl.pallas_call(
        body,
        grid=(GT,),
        in_specs=[
            pl.BlockSpec((NC, BT, 16), lambda i: (0, i, 0)),
            pl.BlockSpec((BT, DIN), lambda i: (i, 0)),
            pl.BlockSpec((DIN, HID), lambda i: (0, 0)),
        ],
        out_specs=[
            pl.BlockSpec((BT, HID), lambda i: (i, 0)),
            pl.BlockSpec((BT, HID), lambda i: (i, 0)),
        ],
        out_shape=[
            jax.ShapeDtypeStruct((N, HID), jnp.float32),
            jax.ShapeDtypeStruct((N, HID), jnp.float32),
        ],
    )(degp, x, W1)


def _tc_mid(accp, hhat, dinv, b2d, Wn):
    def body(accp_ref, hhat_ref, dinv_ref, b_ref, w_ref, h_ref, hn_ref):
        t = (accp_ref[0] + accp_ref[1] + hhat_ref[...]) * dinv_ref[...] \
            + b_ref[...]
        h = jnp.maximum(t, 0.0)
        h_ref[...] = h
        hn_ref[...] = (
            jnp.dot(h, w_ref[...], preferred_element_type=jnp.float32)
            * dinv_ref[...]
        )

    return pl.pallas_call(
        body,
        grid=(GT,),
        in_specs=[
            pl.BlockSpec((NC, BT, HID), lambda i: (0, i, 0)),
            pl.BlockSpec((BT, HID), lambda i: (i, 0)),
            pl.BlockSpec((BT, HID), lambda i: (i, 0)),
            pl.BlockSpec((1, HID), lambda i: (0, 0)),
            pl.BlockSpec((HID, HID), lambda i: (0, 0)),
        ],
        out_specs=[
            pl.BlockSpec((BT, HID), lambda i: (i, 0)),
            pl.BlockSpec((BT, HID), lambda i: (i, 0)),
        ],
        out_shape=[
            jax.ShapeDtypeStruct((N, HID), jnp.float32),
            jax.ShapeDtypeStruct((N, HID), jnp.float32),
        ],
    )(accp, hhat, dinv, b2d, Wn)


def _tc_fin(accp, hhat3, dinv, b2d, h1, h2, Wl, bl2d):
    def body(accp_ref, hhat_ref, dinv_ref, b_ref, h1_ref, h2_ref, wl_ref,
             bl_ref, out_ref):
        t = (accp_ref[0] + accp_ref[1] + hhat_ref[...]) * dinv_ref[...] \
            + b_ref[...]
        h3 = jnp.maximum(t, 0.0)
        acc = jnp.dot(h1_ref[...], wl_ref[0:HID, :],
                      preferred_element_type=jnp.float32)
        acc += jnp.dot(h2_ref[...], wl_ref[HID:2 * HID, :],
                       preferred_element_type=jnp.float32)
        acc += jnp.dot(h3, wl_ref[2 * HID:3 * HID, :],
                       preferred_element_type=jnp.float32)
        out_ref[...] = acc + bl_ref[...]

    return pl.pallas_call(
        body,
        grid=(GT,),
        in_specs=[
            pl.BlockSpec((NC, BT, HID), lambda i: (0, i, 0)),
            pl.BlockSpec((BT, HID), lambda i: (i, 0)),
            pl.BlockSpec((BT, HID), lambda i: (i, 0)),
            pl.BlockSpec((1, HID), lambda i: (0, 0)),
            pl.BlockSpec((BT, HID), lambda i: (i, 0)),
            pl.BlockSpec((BT, HID), lambda i: (i, 0)),
            pl.BlockSpec((3 * HID, DOUT), lambda i: (0, 0)),
            pl.BlockSpec((1, DOUT), lambda i: (0, 0)),
        ],
        out_specs=pl.BlockSpec((BT, DOUT), lambda i: (i, 0)),
        out_shape=jax.ShapeDtypeStruct((N, DOUT), jnp.float32),
    )(accp, hhat3, dinv, b2d, h1, h2, Wl, bl2d)


def kernel(x, edge_index, W1, b1, W2, b2, W3, b3, Wl, bl):
    src3 = edge_index[0].reshape(NW, CH, K)
    dst3 = edge_index[1].reshape(NW, CH, K)

    degp = _sc_degree(dst3)
    dinv, hhat1 = _tc_pre(degp, x, W1)

    acc1 = _sc_propagate(hhat1, src3, dst3)
    h1, hhat2 = _tc_mid(acc1, hhat1, dinv, b1.reshape(1, HID), W2)

    acc2 = _sc_propagate(hhat2, src3, dst3)
    h2, hhat3 = _tc_mid(acc2, hhat2, dinv, b2.reshape(1, HID), W3)

    acc3 = _sc_propagate(hhat3, src3, dst3)
    out = _tc_fin(acc3, hhat3, dinv, b3.reshape(1, HID), h1, h2, Wl,
                  bl.reshape(1, DOUT))
    return out


# speedup vs baseline: 1.0692x; 1.0503x over previous
import functools

import jax
import jax.numpy as jnp
from jax import lax
from jax.experimental import pallas as pl
from jax.experimental.pallas import tpu as pltpu
from jax.experimental.pallas import tpu_sc as plsc

N = 10000
E = 320000
DIN = 128
HID = 128
DOUT = 64

NC = 2
NS = 16
NW = NC * NS
K = 80
EW = E // NW
CH = EW // K
CI = 63
NB = 4
RZC = 80
NRC = N // RZC
NZJ = -(-NRC // NS)

BT = 1000
GT = N // BT


def _mesh():
    return plsc.VectorSubcoreMesh(
        core_axis_name="c", subcore_axis_name="s", num_cores=NC, num_subcores=NS
    )


def _sc_degree(dst3):
    @functools.partial(
        pl.kernel,
        out_type=jax.ShapeDtypeStruct((NC, N, 16), jnp.float32),
        mesh=_mesh(),
        scratch_types=[
            pltpu.VMEM((CH, K), jnp.int32),
            pltpu.VMEM((K, 16), jnp.float32),
            pltpu.VMEM((RZC, 16), jnp.float32),
            pltpu.VMEM_SHARED((N, 16), jnp.float32),
            pltpu.SemaphoreType.DMA,
        ],
        compiler_params=pltpu.CompilerParams(use_tc_tiling_on_sc=False),
    )
    def deg_kernel(dst_hbm, out_hbm, didx_v, ones_v, zbuf_v, acc_sh, isem):
        cid = lax.axis_index("c")
        sid = lax.axis_index("s")
        wid = cid * NS + sid
        di = pltpu.async_copy(dst_hbm.at[wid], didx_v, isem)

        def fill(i, carry):
            ones_v[i, :] = jnp.full((16,), 1.0, jnp.float32)
            return carry

        lax.fori_loop(0, K, fill, 0)

        def zrow(i, carry):
            zbuf_v[i, :] = jnp.zeros((16,), jnp.float32)
            return carry

        lax.fori_loop(0, RZC, zrow, 0)

        def zchunk(j, carry):
            c = sid + j * NS

            @pl.when(c < NRC)
            def _():
                pltpu.sync_copy(zbuf_v, acc_sh.at[pl.ds(c * RZC, RZC)])

            return carry

        lax.fori_loop(0, NZJ, zchunk, 0)
        di.wait()
        plsc.subcore_barrier()

        def chunk(c, carry):
            pltpu.sync_copy(ones_v, acc_sh.at[didx_v.at[c]], add=True)
            return carry

        lax.fori_loop(0, CH, chunk, 0)
        plsc.subcore_barrier()

        def ochunk(j, carry):
            c = sid + j * NS

            @pl.when(c < NRC)
            def _():
                r0 = c * RZC
                pltpu.sync_copy(acc_sh.at[pl.ds(r0, RZC)],
                                out_hbm.at[cid, pl.ds(r0, RZC)])

            return carry

        lax.fori_loop(0, NZJ, ochunk, 0)

    return deg_kernel(dst3)


def _sc_propagate(hhat, src3, dst3):
    @functools.partial(
        pl.kernel,
        out_type=jax.ShapeDtypeStruct((NC, N, HID), jnp.float32),
        mesh=_mesh(),
        scratch_types=[
            pltpu.VMEM((CI, K), jnp.int32),
            pltpu.VMEM((CI, K), jnp.int32),
            [pltpu.VMEM((K, HID), jnp.float32)] * NB,
            pltpu.VMEM_SHARED((N, HID), jnp.float32),
            [pltpu.SemaphoreType.DMA] * NB,
            [pltpu.SemaphoreType.DMA] * NB,
        ],
        compiler_params=pltpu.CompilerParams(use_tc_tiling_on_sc=False),
    )
    def prop_kernel(hhat_hbm, src_hbm, dst_hbm, out_hbm,
                    sidx_v, didx_v, rows_bufs, acc_sh, gsems, ssems):
        zbuf_v = rows_bufs[0]
        cid = lax.axis_index("c")
        sid = lax.axis_index("s")
        wid = cid * NS + sid
        dsi = pltpu.async_copy(src_hbm.at[wid, pl.ds(0, CI)], sidx_v, gsems[0])
        ddi = pltpu.async_copy(dst_hbm.at[wid, pl.ds(0, CI)], didx_v, gsems[1])

        def zrow(i, carry):
            for j in range(HID // 16):
                zbuf_v[i, pl.ds(j * 16, 16)] = jnp.zeros((16,), jnp.float32)
            return carry

        lax.fori_loop(0, RZC, zrow, 0)

        def zchunk(j, carry):
            c = sid + j * NS

            @pl.when(c < NRC)
            def _():
                pltpu.sync_copy(zbuf_v, acc_sh.at[pl.ds(c * RZC, RZC)])

            return carry

        lax.fori_loop(0, NZJ, zchunk, 0)
        dsi.wait()
        ddi.wait()
        plsc.subcore_barrier()

        def run_phase(count):
            ng = count // NB
            tail = count - ng * NB

            for j in range(NB):
                pltpu.async_copy(hhat_hbm.at[sidx_v.at[j]], rows_bufs[j],
                                 gsems[j])

            def chunk_group(g, carry):
                c = g * NB
                sdescs = []
                for j in range(NB):
                    pltpu.make_async_copy(hhat_hbm.at[sidx_v.at[c + j]],
                                          rows_bufs[j], gsems[j]).wait()
                    sdescs.append(
                        pltpu.async_copy(rows_bufs[j],
                                         acc_sh.at[didx_v.at[c + j]],
                                         ssems[j], add=True))
                for j in range(NB):
                    sdescs[j].wait()

                    @pl.when(g < ng - 1)
                    def _():
                        pltpu.async_copy(hhat_hbm.at[sidx_v.at[c + NB + j]],
                                         rows_bufs[j], gsems[j])

                return carry

            lax.fori_loop(0, ng, chunk_group, 0)
            for t in range(tail):
                c = ng * NB + t
                pltpu.async_copy(hhat_hbm.at[sidx_v.at[c]], rows_bufs[t],
                                 gsems[t]).wait()
                pltpu.sync_copy(rows_bufs[t], acc_sh.at[didx_v.at[c]],
                                add=True)

        run_phase(CI)
        pltpu.sync_copy(src_hbm.at[wid, pl.ds(CI, CH - CI)],
                        sidx_v.at[pl.ds(0, CH - CI)])
        pltpu.sync_copy(dst_hbm.at[wid, pl.ds(CI, CH - CI)],
                        didx_v.at[pl.ds(0, CH - CI)])
        run_phase(CH - CI)
        plsc.subcore_barrier()

        def ochunk(j, carry):
            c = sid + j * NS

            @pl.when(c < NRC)
            def _():
                r0 = c * RZC
                pltpu.sync_copy(acc_sh.at[pl.ds(r0, RZC)],
                                out_hbm.at[cid, pl.ds(r0, RZC)])

            return carry

        lax.fori_loop(0, NZJ, ochunk, 0)

    return prop_kernel(hhat, src3, dst3)


def _tc_pre(degp, x, W1):
    def body(degp_ref, x_ref, w_ref, dinv_ref, hhat_ref):
        deg = degp_ref[0, :, 0:1] + degp_ref[1, :, 0:1] + 1.0
        dinv = lax.rsqrt(jnp.maximum(deg, 1.0))
        dinvb = jnp.broadcast_to(dinv, (BT, HID))
        dinv_ref[...] = dinvb
        hhat_ref[...] = (
            jnp.dot(x_ref[...], w_ref[...], preferred_element_type=jnp.float32)
            * dinvb
        )

    return pl.pallas_call(
        body,
        grid=(GT,),
        in_specs=[
            pl.BlockSpec((NC, BT, 16), lambda i: (0, i, 0)),
            pl.BlockSpec((BT, DIN), lambda i: (i, 0)),
            pl.BlockSpec((DIN, HID), lambda i: (0, 0)),
        ],
        out_specs=[
            pl.BlockSpec((BT, HID), lambda i: (i, 0)),
            pl.BlockSpec((BT, HID), lambda i: (i, 0)),
        ],
        out_shape=[
            jax.ShapeDtypeStruct((N, HID), jnp.float32),
            jax.ShapeDtypeStruct((N, HID), jnp.float32),
        ],
    )(degp, x, W1)


def _tc_mid(accp, hhat, dinv, b2d, Wn):
    def body(accp_ref, hhat_ref, dinv_ref, b_ref, w_ref, h_ref, hn_ref):
        t = (accp_ref[0] + accp_ref[1] + hhat_ref[...]) * dinv_ref[...] \
            + b_ref[...]
        h = jnp.maximum(t, 0.0)
        h_ref[...] = h
        hn_ref[...] = (
            jnp.dot(h, w_ref[...], preferred_element_type=jnp.float32)
            * dinv_ref[...]
        )

    return pl.pallas_call(
        body,
        grid=(GT,),
        in_specs=[
            pl.BlockSpec((NC, BT, HID), lambda i: (0, i, 0)),
            pl.BlockSpec((BT, HID), lambda i: (i, 0)),
            pl.BlockSpec((BT, HID), lambda i: (i, 0)),
            pl.BlockSpec((1, HID), lambda i: (0, 0)),
            pl.BlockSpec((HID, HID), lambda i: (0, 0)),
        ],
        out_specs=[
            pl.BlockSpec((BT, HID), lambda i: (i, 0)),
            pl.BlockSpec((BT, HID), lambda i: (i, 0)),
        ],
        out_shape=[
            jax.ShapeDtypeStruct((N, HID), jnp.float32),
            jax.ShapeDtypeStruct((N, HID), jnp.float32),
        ],
    )(accp, hhat, dinv, b2d, Wn)


def _tc_fin(accp, hhat3, dinv, b2d, h1, h2, Wl, bl2d):
    def body(accp_ref, hhat_ref, dinv_ref, b_ref, h1_ref, h2_ref, wl_ref,
             bl_ref, out_ref):
        t = (accp_ref[0] + accp_ref[1] + hhat_ref[...]) * dinv_ref[...] \
            + b_ref[...]
        h3 = jnp.maximum(t, 0.0)
        acc = jnp.dot(h1_ref[...], wl_ref[0:HID, :],
                      preferred_element_type=jnp.float32)
        acc += jnp.dot(h2_ref[...], wl_ref[HID:2 * HID, :],
                       preferred_element_type=jnp.float32)
        acc += jnp.dot(h3, wl_ref[2 * HID:3 * HID, :],
                       preferred_element_type=jnp.float32)
        out_ref[...] = acc + bl_ref[...]

    return pl.pallas_call(
        body,
        grid=(GT,),
        in_specs=[
            pl.BlockSpec((NC, BT, HID), lambda i: (0, i, 0)),
            pl.BlockSpec((BT, HID), lambda i: (i, 0)),
            pl.BlockSpec((BT, HID), lambda i: (i, 0)),
            pl.BlockSpec((1, HID), lambda i: (0, 0)),
            pl.BlockSpec((BT, HID), lambda i: (i, 0)),
            pl.BlockSpec((BT, HID), lambda i: (i, 0)),
            pl.BlockSpec((3 * HID, DOUT), lambda i: (0, 0)),
            pl.BlockSpec((1, DOUT), lambda i: (0, 0)),
        ],
        out_specs=pl.BlockSpec((BT, DOUT), lambda i: (i, 0)),
        out_shape=jax.ShapeDtypeStruct((N, DOUT), jnp.float32),
    )(accp, hhat3, dinv, b2d, h1, h2, Wl, bl2d)


def kernel(x, edge_index, W1, b1, W2, b2, W3, b3, Wl, bl):
    src3 = edge_index[0].reshape(NW, CH, K)
    dst3 = edge_index[1].reshape(NW, CH, K)

    degp = _sc_degree(dst3)
    dinv, hhat1 = _tc_pre(degp, x, W1)

    acc1 = _sc_propagate(hhat1, src3, dst3)
    h1, hhat2 = _tc_mid(acc1, hhat1, dinv, b1.reshape(1, HID), W2)

    acc2 = _sc_propagate(hhat2, src3, dst3)
    h2, hhat3 = _tc_mid(acc2, hhat2, dinv, b2.reshape(1, HID), W3)

    acc3 = _sc_propagate(hhat3, src3, dst3)
    out = _tc_fin(acc3, hhat3, dinv, b3.reshape(1, HID), h1, h2, Wl,
                  bl.reshape(1, DOUT))
    return out
